# baseline (speedup 1.0000x reference)
import jax
import jax.numpy as jnp
from jax.experimental import pallas as pl
from jax.experimental.pallas import tpu as pltpu

_N = 512
_EPS = 1e-5


def _mm(a, b):
    return jax.lax.dot_general(a, b, (((1,), (0,)), ((), ())),
                               preferred_element_type=jnp.float32)


def _mmt(a, bt):
    return jax.lax.dot_general(a, bt, (((1,), (1,)), ((), ())),
                               preferred_element_type=jnp.float32)


def _mmh(a, b):
    return jax.lax.dot_general(a, b, (((1,), (0,)), ((), ())),
                               preferred_element_type=jnp.float32,
                               precision=jax.lax.Precision.HIGHEST)


def _fused(x_ref, wggl_ref, bggl_ref,
           c1w1_h, c1w2a_h, c1w2b_h, c1w3a_h, c1w3b_h, c1w3c_h,
           c1b1_ref, c1b2_ref, c1b3_ref,
           c2w1_h, c2w2a_h, c2w2b_h, c2w3a_h, c2w3b_h, c2w3c_h,
           c2b1_ref, c2b2_ref, c2b3_ref,
           bn1g_ref, bn1b_ref, bn2g_ref, bn2b_ref,
           w5_h, b5_ref, out_ref,
           c1w1_v, c1w2a_v, c1w2b_v, c1w3a_v, c1w3b_v, c1w3c_v,
           c2w1_v, c2w2a_v, c2w2b_v, c2w3a_v, c2w3b_v, c2w3c_v,
           w5_v, sem):
    n = _N
    srcs = [c1w1_h, c1w2a_h, c1w3a_h, c1w2b_h, c1w3b_h, c1w3c_h,
            c2w1_h, c2w2b_h, c2w2a_h, c2w3b_h, c2w3c_h, c2w3a_h, w5_h]
    dsts = [c1w1_v, c1w2a_v, c1w3a_v, c1w2b_v, c1w3b_v, c1w3c_v,
            c2w1_v, c2w2b_v, c2w2a_v, c2w3b_v, c2w3c_v, c2w3a_v, w5_v]
    copies = [pltpu.make_async_copy(s, d, sem.at[i])
              for i, (s, d) in enumerate(zip(srcs, dsts))]
    for cp in copies:
        cp.start()

    x2 = x_ref[...]
    atrr = jax.nn.sigmoid(_mmt(x2, wggl_ref[...]) + bggl_ref[...])
    a = jax.lax.dot_general(atrr, atrr, (((1,), (1,)), ((), ())),
                            preferred_element_type=jnp.float32)

    copies[0].wait()
    copies[1].wait()
    copies[2].wait()
    h1 = _mmt(x2, c1w1_v[...]) + c1b1_ref[...]
    h2a = _mmt(x2, c1w2a_v[...])
    h3a = _mmt(x2, c1w3a_v[...])

    maxval = jnp.max(a, axis=1, keepdims=True)
    row = jax.lax.broadcasted_iota(jnp.int32, (n, n), 0)
    col = jax.lax.broadcasted_iota(jnp.int32, (n, n), 1)
    am = a * (row != col).astype(jnp.float32)
    inv_max = 1.0 / maxval
    deg = _mmh(am, inv_max)
    dis = jnp.where(deg > 0.0, jax.lax.rsqrt(deg), 0.0)
    lhatT = am * (-(dis * inv_max)) * jnp.transpose(dis)

    t1 = _mm(lhatT, x2)
    copies[3].wait()
    copies[4].wait()
    copies[5].wait()
    h2 = h2a + _mmt(t1, c1w2b_v[...]) + c1b2_ref[...]
    h3b = _mmt(t1, c1w3b_v[...])
    t2 = 2.0 * _mm(lhatT, t1) - x2
    h3 = h3a + h3b + _mmt(t2, c1w3c_v[...]) + c1b3_ref[...]
    h = jnp.concatenate([h1, h2, h3], axis=1)

    ones_row = jnp.full((1, n), 1.0 / n, jnp.float32)
    mu = _mmh(ones_row, h)
    var = _mmh(ones_row, h * h) - mu * mu
    h = (h - mu) * jax.lax.rsqrt(var + _EPS) * bn1g_ref[...] + bn1b_ref[...]

    for cp in copies[6:12]:
        cp.wait()
    g1 = _mmt(h, c2w1_v[...]) + c2b1_ref[...]
    u2 = _mmt(h, c2w2b_v[...])
    g2 = _mmt(h, c2w2a_v[...]) + _mm(lhatT, u2) + c2b2_ref[...]
    u3 = _mmt(h, c2w3b_v[...])
    p3 = _mmt(h, c2w3c_v[...])
    q3 = _mm(lhatT, p3)
    g3 = (_mmt(h, c2w3a_v[...]) + _mm(lhatT, u3)
          + 2.0 * _mm(lhatT, q3) - p3 + c2b3_ref[...])
    g = jnp.concatenate([g1, g2, g3], axis=1)

    mu2 = _mmh(ones_row, g)
    var2 = _mmh(ones_row, g * g) - mu2 * mu2
    alpha = jax.lax.rsqrt(var2 + _EPS) * bn2g_ref[...]
    beta = bn2b_ref[...] - mu2 * alpha
    copies[12].wait()
    w5 = w5_v[...]
    out_ref[...] = jnp.maximum(
        _mm(g, jnp.transpose(alpha) * w5) + (_mm(beta, w5) + b5_ref[...]),
        0.0)


_HBM = pl.BlockSpec(memory_space=pl.ANY)
_VMEM = pl.BlockSpec(memory_space=pltpu.MemorySpace.VMEM)


@jax.jit
def kernel(x, w_ggl, b_ggl, c1w1, c1w2a, c1w2b, c1w3a, c1w3b, c1w3c,
           c1b1, c1b2, c1b3, c2w1, c2w2a, c2w2b, c2w3a, c2w3b, c2w3c,
           c2b1, c2b2, c2b3, bn1_g, bn1_b, bn2_g, bn2_b, w5, b5):
    f32 = jnp.float32
    return pl.pallas_call(
        _fused,
        out_shape=jax.ShapeDtypeStruct((_N, 256), f32),
        in_specs=[_VMEM, _VMEM, _VMEM,
                  _HBM, _HBM, _HBM, _HBM, _HBM, _HBM,
                  _VMEM, _VMEM, _VMEM,
                  _HBM, _HBM, _HBM, _HBM, _HBM, _HBM,
                  _VMEM, _VMEM, _VMEM,
                  _VMEM, _VMEM, _VMEM, _VMEM,
                  _HBM, _VMEM],
        scratch_shapes=(
            [pltpu.VMEM((400, 256), f32)] * 6
            + [pltpu.VMEM((100, 1200), f32)] * 6
            + [pltpu.VMEM((300, 256), f32),
               pltpu.SemaphoreType.DMA((13,))]),
    )(x, w_ggl.T, b_ggl,
      c1w1.T, c1w2a.T, c1w2b.T, c1w3a.T, c1w3b.T, c1w3c.T,
      c1b1, c1b2, c1b3,
      c2w1.T, c2w2a.T, c2w2b.T, c2w3a.T, c2w3b.T, c2w3c.T,
      c2b1, c2b2, c2b3, bn1_g, bn1_b, bn2_g, bn2_b, w5, b5)


# speedup vs baseline: 1.5581x; 1.5581x over previous
import jax
import jax.numpy as jnp
from jax.experimental import pallas as pl
from jax.experimental.pallas import tpu as pltpu

_N = 512
_EPS = 1e-5


def _mm(a, b):
    return jax.lax.dot_general(a, b, (((1,), (0,)), ((), ())),
                               preferred_element_type=jnp.float32)


def _mmt(a, bt):
    return jax.lax.dot_general(a, bt, (((1,), (1,)), ((), ())),
                               preferred_element_type=jnp.float32)


def _fused(x_ref, wggl_ref, bggl_ref,
           c1w1_h, c1w2a_h, c1w2b_h, c1w3a_h, c1w3b_h, c1w3c_h,
           c1b1_ref, c1b2_ref, c1b3_ref,
           c2w1_h, c2w2a_h, c2w2b_h, c2w3a_h, c2w3b_h, c2w3c_h,
           c2b1_ref, c2b2_ref, c2b3_ref,
           bn1g_ref, bn1b_ref, bn2g_ref, bn2b_ref,
           w5_h, b5_ref, out_ref,
           c1w1_v, c1w2a_v, c1w2b_v, c1w3a_v, c1w3b_v, c1w3c_v,
           c2w1_v, c2w2a_v, c2w2b_v, c2w3a_v, c2w3b_v, c2w3c_v,
           w5_v, sem):
    n = _N
    srcs = [c1w1_h, c1w2a_h, c1w2b_h, c1w3a_h, c1w3b_h, c1w3c_h,
            c2w1_h, c2w2a_h, c2w2b_h, c2w3a_h, c2w3b_h, c2w3c_h, w5_h]
    dsts = [c1w1_v, c1w2a_v, c1w2b_v, c1w3a_v, c1w3b_v, c1w3c_v,
            c2w1_v, c2w2a_v, c2w2b_v, c2w3a_v, c2w3b_v, c2w3c_v, w5_v]
    copies = [pltpu.make_async_copy(s, d, sem.at[i])
              for i, (s, d) in enumerate(zip(srcs, dsts))]
    for cp in copies:
        cp.start()

    x2 = x_ref[...]
    atrr = jax.nn.sigmoid(_mmt(x2, wggl_ref[...]) + bggl_ref[...])
    a = jax.lax.dot_general(atrr, atrr, (((1,), (1,)), ((), ())),
                            preferred_element_type=jnp.float32)
    maxval = jnp.max(a, axis=1, keepdims=True)
    row = jax.lax.broadcasted_iota(jnp.int32, (n, n), 0)
    col = jax.lax.broadcasted_iota(jnp.int32, (n, n), 1)
    offdiag = (row != col).astype(jnp.float32)
    inv_max = 1.0 / maxval
    adj = a * offdiag * jnp.transpose(inv_max)
    deg = jnp.sum(adj, axis=1, keepdims=True)
    dis = jnp.where(deg > 0.0, jax.lax.rsqrt(deg), 0.0)
    lhatT = (a * offdiag) * (-(dis * inv_max)) * jnp.transpose(dis)

    t1 = _mm(lhatT, x2)
    t2 = 2.0 * _mm(lhatT, t1) - x2

    for cp in copies[:6]:
        cp.wait()
    h1 = _mmt(x2, c1w1_v[...]) + c1b1_ref[...]
    h2 = _mmt(x2, c1w2a_v[...]) + _mmt(t1, c1w2b_v[...]) + c1b2_ref[...]
    h3 = (_mmt(x2, c1w3a_v[...]) + _mmt(t1, c1w3b_v[...])
          + _mmt(t2, c1w3c_v[...]) + c1b3_ref[...])
    h = jnp.concatenate([h1, h2, h3], axis=1)

    mu = jnp.mean(h, axis=0, keepdims=True)
    var = jnp.mean(h * h, axis=0, keepdims=True) - mu * mu
    h = (h - mu) * jax.lax.rsqrt(var + _EPS) * bn1g_ref[...] + bn1b_ref[...]

    for cp in copies[6:12]:
        cp.wait()
    g1 = _mmt(h, c2w1_v[...]) + c2b1_ref[...]
    u2 = _mmt(h, c2w2b_v[...])
    g2 = _mmt(h, c2w2a_v[...]) + _mm(lhatT, u2) + c2b2_ref[...]
    u3 = _mmt(h, c2w3b_v[...])
    p3 = _mmt(h, c2w3c_v[...])
    q3 = _mm(lhatT, p3)
    g3 = (_mmt(h, c2w3a_v[...]) + _mm(lhatT, u3)
          + 2.0 * _mm(lhatT, q3) - p3 + c2b3_ref[...])
    g = jnp.concatenate([g1, g2, g3], axis=1)

    mu2 = jnp.mean(g, axis=0, keepdims=True)
    var2 = jnp.mean(g * g, axis=0, keepdims=True) - mu2 * mu2
    alpha = jax.lax.rsqrt(var2 + _EPS) * bn2g_ref[...]
    beta = bn2b_ref[...] - mu2 * alpha
    copies[12].wait()
    w5 = w5_v[...]
    out_ref[...] = jnp.maximum(
        _mm(g, jnp.transpose(alpha) * w5) + (_mm(beta, w5) + b5_ref[...]),
        0.0)


_HBM = pl.BlockSpec(memory_space=pl.ANY)
_VMEM = pl.BlockSpec(memory_space=pltpu.MemorySpace.VMEM)


@jax.jit
def kernel(x, w_ggl, b_ggl, c1w1, c1w2a, c1w2b, c1w3a, c1w3b, c1w3c,
           c1b1, c1b2, c1b3, c2w1, c2w2a, c2w2b, c2w3a, c2w3b, c2w3c,
           c2b1, c2b2, c2b3, bn1_g, bn1_b, bn2_g, bn2_b, w5, b5):
    f32 = jnp.float32
    return pl.pallas_call(
        _fused,
        out_shape=jax.ShapeDtypeStruct((_N, 256), f32),
        in_specs=[_VMEM, _VMEM, _VMEM,
                  _HBM, _HBM, _HBM, _HBM, _HBM, _HBM,
                  _VMEM, _VMEM, _VMEM,
                  _HBM, _HBM, _HBM, _HBM, _HBM, _HBM,
                  _VMEM, _VMEM, _VMEM,
                  _VMEM, _VMEM, _VMEM, _VMEM,
                  _HBM, _VMEM],
        scratch_shapes=(
            [pltpu.VMEM((400, 256), f32)] * 6
            + [pltpu.VMEM((100, 1200), f32)] * 6
            + [pltpu.VMEM((300, 256), f32),
               pltpu.SemaphoreType.DMA((13,))]),
    )(x, w_ggl.T, b_ggl,
      c1w1.T, c1w2a.T, c1w2b.T, c1w3a.T, c1w3b.T, c1w3c.T,
      c1b1, c1b2, c1b3,
      c2w1.T, c2w2a.T, c2w2b.T, c2w3a.T, c2w3b.T, c2w3c.T,
      c2b1, c2b2, c2b3, bn1_g, bn1_b, bn2_g, bn2_b, w5, b5)


# speedup vs baseline: 1.5630x; 1.0031x over previous
import jax
import jax.numpy as jnp
from jax.experimental import pallas as pl
from jax.experimental.pallas import tpu as pltpu

_N = 512
_EPS = 1e-5


def _mm(a, b):
    return jax.lax.dot_general(a, b, (((1,), (0,)), ((), ())),
                               preferred_element_type=jnp.float32)


def _mmt(a, bt):
    return jax.lax.dot_general(a, bt, (((1,), (1,)), ((), ())),
                               preferred_element_type=jnp.float32)


def _fused(x_ref, wggl_ref, bggl_ref,
           c1w1_h, c1w2a_h, c1w2b_h, c1w3a_h, c1w3b_h, c1w3c_h,
           c1b1_h, c1b2_h, c1b3_h,
           c2w1_h, c2w2a_h, c2w2b_h, c2w3a_h, c2w3b_h, c2w3c_h,
           c2b1_h, c2b2_h, c2b3_h,
           bn1g_h, bn1b_h, bn2g_h, bn2b_h,
           w5_h, b5_h, out_ref,
           c1w1_v, c1w2a_v, c1w2b_v, c1w3a_v, c1w3b_v, c1w3c_v,
           c1b1_v, c1b2_v, c1b3_v,
           c2w1_v, c2w2a_v, c2w2b_v, c2w3a_v, c2w3b_v, c2w3c_v,
           c2b1_v, c2b2_v, c2b3_v,
           bn1g_v, bn1b_v, bn2g_v, bn2b_v,
           w5_v, b5_v, sem):
    n = _N
    srcs = [c1w1_h, c1w2a_h, c1w2b_h, c1w3a_h, c1w3b_h, c1w3c_h,
            c1b1_h, c1b2_h, c1b3_h, bn1g_h, bn1b_h,
            c2w1_h, c2w2a_h, c2w2b_h, c2w3a_h, c2w3b_h, c2w3c_h,
            c2b1_h, c2b2_h, c2b3_h, bn2g_h, bn2b_h,
            w5_h, b5_h]
    dsts = [c1w1_v, c1w2a_v, c1w2b_v, c1w3a_v, c1w3b_v, c1w3c_v,
            c1b1_v, c1b2_v, c1b3_v, bn1g_v, bn1b_v,
            c2w1_v, c2w2a_v, c2w2b_v, c2w3a_v, c2w3b_v, c2w3c_v,
            c2b1_v, c2b2_v, c2b3_v, bn2g_v, bn2b_v,
            w5_v, b5_v]
    copies = [pltpu.make_async_copy(s, d, sem.at[i])
              for i, (s, d) in enumerate(zip(srcs, dsts))]
    for cp in copies:
        cp.start()

    x2 = x_ref[...]
    atrr = jax.nn.sigmoid(_mmt(x2, wggl_ref[...]) + bggl_ref[...])
    a = jax.lax.dot_general(atrr, atrr, (((1,), (1,)), ((), ())),
                            preferred_element_type=jnp.float32)
    maxval = jnp.max(a, axis=1, keepdims=True)
    row = jax.lax.broadcasted_iota(jnp.int32, (n, n), 0)
    col = jax.lax.broadcasted_iota(jnp.int32, (n, n), 1)
    offdiag = (row != col).astype(jnp.float32)
    inv_max = 1.0 / maxval
    adj = a * offdiag * jnp.transpose(inv_max)
    deg = jnp.sum(adj, axis=1, keepdims=True)
    dis = jnp.where(deg > 0.0, jax.lax.rsqrt(deg), 0.0)
    lhatT = (a * offdiag) * (-(dis * inv_max)) * jnp.transpose(dis)

    t1 = _mm(lhatT, x2)
    t2 = 2.0 * _mm(lhatT, t1) - x2

    for cp in copies[:11]:
        cp.wait()
    h1 = _mmt(x2, c1w1_v[...]) + c1b1_v[...]
    h2 = _mmt(x2, c1w2a_v[...]) + _mmt(t1, c1w2b_v[...]) + c1b2_v[...]
    h3 = (_mmt(x2, c1w3a_v[...]) + _mmt(t1, c1w3b_v[...])
          + _mmt(t2, c1w3c_v[...]) + c1b3_v[...])
    h = jnp.concatenate([h1, h2, h3], axis=1)

    mu = jnp.mean(h, axis=0, keepdims=True)
    var = jnp.mean(h * h, axis=0, keepdims=True) - mu * mu
    h = (h - mu) * jax.lax.rsqrt(var + _EPS) * bn1g_v[...] + bn1b_v[...]

    for cp in copies[11:22]:
        cp.wait()
    g1 = _mmt(h, c2w1_v[...]) + c2b1_v[...]
    u2 = _mmt(h, c2w2b_v[...])
    g2 = _mmt(h, c2w2a_v[...]) + _mm(lhatT, u2) + c2b2_v[...]
    u3 = _mmt(h, c2w3b_v[...])
    p3 = _mmt(h, c2w3c_v[...])
    q3 = _mm(lhatT, p3)
    g3 = (_mmt(h, c2w3a_v[...]) + _mm(lhatT, u3)
          + 2.0 * _mm(lhatT, q3) - p3 + c2b3_v[...])
    g = jnp.concatenate([g1, g2, g3], axis=1)

    mu2 = jnp.mean(g, axis=0, keepdims=True)
    var2 = jnp.mean(g * g, axis=0, keepdims=True) - mu2 * mu2
    g = (g - mu2) * jax.lax.rsqrt(var2 + _EPS) * bn2g_v[...] + bn2b_v[...]

    copies[22].wait()
    copies[23].wait()
    out_ref[...] = jnp.maximum(_mm(g, w5_v[...]) + b5_v[...], 0.0)


_HBM = pl.BlockSpec(memory_space=pl.ANY)
_VMEM = pl.BlockSpec(memory_space=pltpu.MemorySpace.VMEM)


@jax.jit
def kernel(x, w_ggl, b_ggl, c1w1, c1w2a, c1w2b, c1w3a, c1w3b, c1w3c,
           c1b1, c1b2, c1b3, c2w1, c2w2a, c2w2b, c2w3a, c2w3b, c2w3c,
           c2b1, c2b2, c2b3, bn1_g, bn1_b, bn2_g, bn2_b, w5, b5):
    f32 = jnp.float32
    return pl.pallas_call(
        _fused,
        out_shape=jax.ShapeDtypeStruct((_N, 256), f32),
        in_specs=[_VMEM, _VMEM, _VMEM] + [_HBM] * 24,
        scratch_shapes=(
            [pltpu.VMEM((400, 256), f32)] * 6
            + [pltpu.VMEM((400,), f32)] * 3
            + [pltpu.VMEM((100, 1200), f32)] * 6
            + [pltpu.VMEM((100,), f32)] * 3
            + [pltpu.VMEM((1200,), f32)] * 2
            + [pltpu.VMEM((300,), f32)] * 2
            + [pltpu.VMEM((300, 256), f32),
               pltpu.VMEM((256,), f32),
               pltpu.SemaphoreType.DMA((24,))]),
    )(x, w_ggl.T, b_ggl,
      c1w1.T, c1w2a.T, c1w2b.T, c1w3a.T, c1w3b.T, c1w3c.T,
      c1b1, c1b2, c1b3,
      c2w1.T, c2w2a.T, c2w2b.T, c2w3a.T, c2w3b.T, c2w3c.T,
      c2b1, c2b2, c2b3, bn1_g, bn1_b, bn2_g, bn2_b, w5, b5)
